# baseline (speedup 1.0000x reference)
import functools
import jax
import jax.numpy as jnp
from jax import lax
from jax.experimental import pallas as pl
from jax.experimental.pallas import tpu as pltpu
from jax.experimental.pallas import tpu_sc as plsc

EXP = 8
D = 1024
O = 1024
TOK = 4096
T = 128
NPAD = 5120
NT = NPAD // T
NEID = 48
TR = 512
LOSS_SCALE = 3e-06


def _router_body(x_ref, wr_ref, br_ref, layer_ref):
    logits = lax.dot_general(x_ref[...], wr_ref[...], (((1,), (1,)), ((), ())),
                             preferred_element_type=jnp.float32)
    logits = logits + br_ref[...]
    m = jnp.max(logits, axis=1, keepdims=True)
    iota = lax.broadcasted_iota(jnp.int32, (TR, EXP), 1)
    cand = jnp.where(logits == m, iota, EXP)
    layer_ref[...] = jnp.min(cand, axis=1)


def _router(xf, Wr, br2):
    return pl.pallas_call(
        _router_body,
        grid=(TOK // TR,),
        in_specs=[
            pl.BlockSpec((TR, D), lambda i: (i, 0)),
            pl.BlockSpec((EXP, D), lambda i: (0, 0)),
            pl.BlockSpec((1, EXP), lambda i: (0, 0)),
        ],
        out_specs=pl.BlockSpec((TR,), lambda i: (i,)),
        out_shape=jax.ShapeDtypeStruct((TOK,), jnp.int32),
    )(xf, Wr, br2)


_TPW = TOK // 16
_SPW = NPAD // 16


def _splat(x):
    return jnp.full((16,), x, jnp.int32)


def _splat_sum(v):
    return _splat(jnp.sum(v))


def _dispatch_body(layer_hbm, pidx_hbm, inv_hbm, eid_hbm, loss_hbm,
                   lay_v, inv_v, tid_v, dest_v, stage_i, stage_f, fill_v,
                   eid_v, cnt_all, cnt_sh):
    wid = lax.axis_index("s")
    iota = lax.iota(jnp.int32, 16)
    zero16 = jnp.zeros((16,), jnp.int32)
    widv = _splat(wid)

    base = wid * _TPW
    pltpu.sync_copy(layer_hbm.at[pl.ds(base, _TPW)], lay_v)

    counts = zero16
    ft = zero16 + TOK
    for i in range(_TPW // 16):
        v = lay_v[pl.ds(i * 16, 16)]
        tid = _splat(base + i * 16) + iota
        for e in range(EXP):
            eq = iota == e
            mask = v == e
            c = _splat_sum(mask.astype(jnp.int32))
            counts = jnp.where(eq, counts + c, counts)
            mn = _splat(jnp.min(jnp.where(mask, tid, TOK)))
            ft = jnp.where(eq, jnp.minimum(ft, mn), ft)
    stage_i[pl.ds(0, 16)] = counts
    stage_i[pl.ds(16, 16)] = ft
    pltpu.sync_copy(stage_i, cnt_sh.at[wid])
    plsc.subcore_barrier()

    pltpu.sync_copy(cnt_sh, cnt_all)
    run = zero16
    mypre = zero16
    fmin = zero16 + TOK
    for w in range(16):
        row = cnt_all[w, pl.ds(0, 16)]
        mypre = jnp.where(widv == w, run, mypre)
        run = run + row
        fmin = jnp.minimum(fmin, cnt_all[w, pl.ds(16, 16)])
    c = run
    pc = (c + (T - 1)) & ~(T - 1)
    ends = plsc.cumsum(pc)
    off = ends - pc
    start = off + mypre
    e8v = _splat_sum(jnp.where(fmin == 0, iota, 0))
    ftf = jnp.where(iota < EXP, fmin, 0)
    ends_s = [_splat_sum(jnp.where(iota == e, ends, 0)) for e in range(EXP)]
    ft_s = [_splat_sum(jnp.where(iota == e, ftf, 0)) for e in range(EXP)]

    fb = wid * _SPW
    for j in range(_SPW // 16):
        s = _splat(fb + j * 16) + iota
        seg = zero16
        for e in range(EXP):
            seg = seg + jnp.where(s >= ends_s[e], 1, 0)
        fill = zero16
        for e in range(EXP):
            fill = jnp.where(seg == e, ft_s[e], fill)
        fill_v[pl.ds(j * 16, 16)] = fill

    @pl.when(wid == 0)
    def _():
        for k in range(NEID // 16):
            s = _splat(k * 16 * T) + iota * T
            seg = zero16
            for e in range(EXP):
                seg = seg + jnp.where(s >= ends_s[e], 1, 0)
            eid_v[pl.ds(k * 16, 16)] = jnp.where(seg >= EXP, e8v, seg)
        pltpu.sync_copy(eid_v, eid_hbm)
        cf = c.astype(jnp.float32)
        cm = jnp.where(iota < EXP, cf - (TOK / EXP), 0.0)
        lsv = cm * cm * (LOSS_SCALE / EXP)
        stage_f[...] = jnp.full((16,), jnp.sum(lsv), jnp.float32)
        pltpu.sync_copy(stage_f, loss_hbm)

    pltpu.sync_copy(fill_v, pidx_hbm.at[pl.ds(fb, _SPW)])
    plsc.subcore_barrier()

    cursors = start
    for i in range(_TPW // 16):
        v = lay_v[pl.ds(i * 16, 16)]
        dest = zero16
        for e in range(EXP):
            mask = v == e
            mi = mask.astype(jnp.int32)
            pcs = plsc.cumsum(mi)
            cnt = _splat_sum(mi)
            cur_e = _splat_sum(jnp.where(iota == e, cursors, 0))
            dest = jnp.where(mask, cur_e + pcs - 1, dest)
            cursors = jnp.where(iota == e, cursors + cnt, cursors)
        inv_v[pl.ds(i * 16, 16)] = dest
        row, col = divmod(i, 8)
        dest_v[row, pl.ds(col * 16, 16)] = dest
        tid_v[row, pl.ds(col * 16, 16)] = _splat(base + i * 16) + iota
    pltpu.sync_copy(inv_v, inv_hbm.at[pl.ds(base, _TPW)])
    for k in range(2):
        pltpu.sync_copy(tid_v.at[k], pidx_hbm.at[dest_v.at[k]])


def _dispatch(layer):
    mesh = plsc.VectorSubcoreMesh(core_axis_name="c", subcore_axis_name="s",
                                  num_cores=1, num_subcores=16)
    f = pl.kernel(
        _dispatch_body,
        out_type=(
            jax.ShapeDtypeStruct((NPAD,), jnp.int32),
            jax.ShapeDtypeStruct((TOK,), jnp.int32),
            jax.ShapeDtypeStruct((NEID,), jnp.int32),
            jax.ShapeDtypeStruct((16,), jnp.float32),
        ),
        mesh=mesh,
        scratch_types=[
            pltpu.VMEM((_TPW,), jnp.int32),
            pltpu.VMEM((_TPW,), jnp.int32),
            pltpu.VMEM((2, 128), jnp.int32),
            pltpu.VMEM((2, 128), jnp.int32),
            pltpu.VMEM((128,), jnp.int32),
            pltpu.VMEM((16,), jnp.float32),
            pltpu.VMEM((_SPW,), jnp.int32),
            pltpu.VMEM((NEID,), jnp.int32),
            pltpu.VMEM((16, 128), jnp.int32),
            pltpu.VMEM_SHARED((16, 128), jnp.int32),
        ],
        compiler_params=pltpu.CompilerParams(needs_layout_passes=False),
    )
    return f(layer)


_GB = 40


def _gatherx_body(pidx_hbm, xf_hbm, xg_hbm, idx_v, rows0, rows1, g0, g1, w0, w1):
    wid = lax.axis_index("s") * 2 + lax.axis_index("c")
    sb = wid * (NPAD // 32)
    pltpu.sync_copy(pidx_hbm.at[pl.ds(sb, NPAD // 32)], idx_v)
    bufs = (rows0, rows1)
    gsem = (g0, g1)
    wsem = (w0, w1)
    gd = [pltpu.make_async_copy(xf_hbm.at[idx_v.at[pl.ds(b * _GB, _GB)]],
                                bufs[b % 2], gsem[b % 2]) for b in range(4)]
    wd = [pltpu.make_async_copy(bufs[b % 2],
                                xg_hbm.at[pl.ds(sb + b * _GB, _GB)],
                                wsem[b % 2]) for b in range(4)]
    gd[0].start()
    gd[1].start()
    for b in range(4):
        gd[b].wait()
        wd[b].start()
        if b + 2 < 4:
            wd[b].wait()
            gd[b + 2].start()
    wd[2].wait()
    wd[3].wait()


def _gatherx(pidx, xf):
    mesh = plsc.VectorSubcoreMesh(core_axis_name="c", subcore_axis_name="s",
                                  num_cores=2, num_subcores=16)
    f = pl.kernel(
        _gatherx_body,
        out_type=jax.ShapeDtypeStruct((NPAD, D), jnp.float32),
        mesh=mesh,
        scratch_types=[
            pltpu.VMEM((NPAD // 32,), jnp.int32),
            pltpu.VMEM((_GB, D), jnp.float32),
            pltpu.VMEM((_GB, D), jnp.float32),
            pltpu.SemaphoreType.DMA,
            pltpu.SemaphoreType.DMA,
            pltpu.SemaphoreType.DMA,
            pltpu.SemaphoreType.DMA,
        ],
    )
    return f(pidx, xf)


_OB = 32


def _outgather_body(inv_hbm, outg_hbm, out_hbm, idx_v, rows0, rows1, g0, g1, w0, w1):
    wid = lax.axis_index("s") * 2 + lax.axis_index("c")
    tb = wid * (TOK // 32)
    pltpu.sync_copy(inv_hbm.at[pl.ds(tb, TOK // 32)], idx_v)
    bufs = (rows0, rows1)
    gsem = (g0, g1)
    wsem = (w0, w1)
    gd = [pltpu.make_async_copy(outg_hbm.at[idx_v.at[pl.ds(b * _OB, _OB)]],
                                bufs[b % 2], gsem[b % 2]) for b in range(4)]
    wd = [pltpu.make_async_copy(bufs[b % 2],
                                out_hbm.at[pl.ds(tb + b * _OB, _OB)],
                                wsem[b % 2]) for b in range(4)]
    gd[0].start()
    gd[1].start()
    for b in range(4):
        gd[b].wait()
        wd[b].start()
        if b + 2 < 4:
            wd[b].wait()
            gd[b + 2].start()
    wd[2].wait()
    wd[3].wait()


def _outgather(inv, out_g):
    mesh = plsc.VectorSubcoreMesh(core_axis_name="c", subcore_axis_name="s",
                                  num_cores=2, num_subcores=16)
    f = pl.kernel(
        _outgather_body,
        out_type=jax.ShapeDtypeStruct((TOK, O), jnp.float32),
        mesh=mesh,
        scratch_types=[
            pltpu.VMEM((TOK // 32,), jnp.int32),
            pltpu.VMEM((_OB, O), jnp.float32),
            pltpu.VMEM((_OB, O), jnp.float32),
            pltpu.SemaphoreType.DMA,
            pltpu.SemaphoreType.DMA,
            pltpu.SemaphoreType.DMA,
            pltpu.SemaphoreType.DMA,
        ],
    )
    return f(inv, out_g)


def _gmm_body(eid_ref, x_ref, w_ref, b_ref, o_ref):
    acc = lax.dot_general(x_ref[...], w_ref[0], (((1,), (1,)), ((), ())),
                          preferred_element_type=jnp.float32)
    o_ref[...] = acc + b_ref[0]


def _gmm(eid, xg, Wp, bp3):
    grid_spec = pltpu.PrefetchScalarGridSpec(
        num_scalar_prefetch=1,
        grid=(NT,),
        in_specs=[
            pl.BlockSpec((T, D), lambda i, eid_ref: (i, 0)),
            pl.BlockSpec((1, O, D), lambda i, eid_ref: (eid_ref[i], 0, 0)),
            pl.BlockSpec((1, 1, O), lambda i, eid_ref: (eid_ref[i], 0, 0)),
        ],
        out_specs=pl.BlockSpec((T, O), lambda i, eid_ref: (i, 0)),
    )
    return pl.pallas_call(
        _gmm_body,
        grid_spec=grid_spec,
        out_shape=jax.ShapeDtypeStruct((NPAD, O), jnp.float32),
    )(eid, xg, Wp, bp3)


@jax.jit
def kernel(x, Wp, bp, Wr, br):
    Bb, Cc, Dd = x.shape
    xf = x.reshape(Bb * Cc, Dd)

    layer = _router(xf, Wr, br.reshape(1, EXP))
    pidx, inv, eid, lossv = _dispatch(layer)
    xg = _gatherx(pidx, xf)
    out_g = _gmm(eid[:NT], xg, Wp, bp.reshape(EXP, 1, O))
    out_flat = _outgather(inv, out_g)
    return out_flat.reshape(Bb, Cc, O), lossv[0]


# speedup vs baseline: 1.0130x; 1.0130x over previous
import functools
import jax
import jax.numpy as jnp
from jax import lax
from jax.experimental import pallas as pl
from jax.experimental.pallas import tpu as pltpu
from jax.experimental.pallas import tpu_sc as plsc

EXP = 8
D = 1024
O = 1024
TOK = 4096
T = 128
NPAD = 5120
NT = NPAD // T
NEID = 48
TR = 512
LOSS_SCALE = 3e-06


def _router_body(x_ref, wr_ref, br_ref, layer_ref):
    logits = lax.dot_general(x_ref[...], wr_ref[...], (((1,), (1,)), ((), ())),
                             preferred_element_type=jnp.float32)
    logits = logits + br_ref[...]
    m = jnp.max(logits, axis=1, keepdims=True)
    iota = lax.broadcasted_iota(jnp.int32, (TR, EXP), 1)
    cand = jnp.where(logits == m, iota, EXP)
    layer_ref[...] = jnp.min(cand, axis=1)


def _router(xf, Wr, br2):
    return pl.pallas_call(
        _router_body,
        grid=(TOK // TR,),
        in_specs=[
            pl.BlockSpec((TR, D), lambda i: (i, 0)),
            pl.BlockSpec((EXP, D), lambda i: (0, 0)),
            pl.BlockSpec((1, EXP), lambda i: (0, 0)),
        ],
        out_specs=pl.BlockSpec((TR,), lambda i: (i,)),
        out_shape=jax.ShapeDtypeStruct((TOK,), jnp.int32),
    )(xf, Wr, br2)


_TPW = TOK // 16
_SPW = NPAD // 16


def _splat(x):
    return jnp.full((16,), x, jnp.int32)


def _splat_sum(v):
    return _splat(jnp.sum(v))


def _dispatch_body(layer_hbm, pidx_hbm, inv_hbm, eid_hbm, loss_hbm,
                   lay_v, inv_v, tid_v, dest_v, stage_i, stage_f, fill_v,
                   eid_v, cnt_all, cnt_sh):
    wid = lax.axis_index("s")
    iota = lax.iota(jnp.int32, 16)
    zero16 = jnp.zeros((16,), jnp.int32)
    widv = _splat(wid)

    base = wid * _TPW
    pltpu.sync_copy(layer_hbm.at[pl.ds(base, _TPW)], lay_v)

    counts = zero16
    ft = zero16 + TOK
    for i in range(_TPW // 16):
        v = lay_v[pl.ds(i * 16, 16)]
        tid = _splat(base + i * 16) + iota
        for e in range(EXP):
            eq = iota == e
            mask = v == e
            c = _splat_sum(mask.astype(jnp.int32))
            counts = jnp.where(eq, counts + c, counts)
            mn = _splat(jnp.min(jnp.where(mask, tid, TOK)))
            ft = jnp.where(eq, jnp.minimum(ft, mn), ft)
    stage_i[pl.ds(0, 16)] = counts
    stage_i[pl.ds(16, 16)] = ft
    pltpu.sync_copy(stage_i, cnt_sh.at[wid])
    plsc.subcore_barrier()

    pltpu.sync_copy(cnt_sh, cnt_all)
    run = zero16
    mypre = zero16
    fmin = zero16 + TOK
    for w in range(16):
        row = cnt_all[w, pl.ds(0, 16)]
        mypre = jnp.where(widv == w, run, mypre)
        run = run + row
        fmin = jnp.minimum(fmin, cnt_all[w, pl.ds(16, 16)])
    c = run
    pc = (c + (T - 1)) & ~(T - 1)
    ends = plsc.cumsum(pc)
    off = ends - pc
    start = off + mypre
    e8v = _splat_sum(jnp.where(fmin == 0, iota, 0))
    ftf = jnp.where(iota < EXP, fmin, 0)
    ends_s = [_splat_sum(jnp.where(iota == e, ends, 0)) for e in range(EXP)]
    ft_s = [_splat_sum(jnp.where(iota == e, ftf, 0)) for e in range(EXP)]

    fb = wid * _SPW
    for j in range(_SPW // 16):
        s = _splat(fb + j * 16) + iota
        seg = zero16
        for e in range(EXP):
            seg = seg + jnp.where(s >= ends_s[e], 1, 0)
        fill = zero16
        for e in range(EXP):
            fill = jnp.where(seg == e, ft_s[e], fill)
        fill_v[pl.ds(j * 16, 16)] = fill

    @pl.when(wid == 0)
    def _():
        for k in range(NEID // 16):
            s = _splat(k * 16 * T) + iota * T
            seg = zero16
            for e in range(EXP):
                seg = seg + jnp.where(s >= ends_s[e], 1, 0)
            eid_v[pl.ds(k * 16, 16)] = jnp.where(seg >= EXP, e8v, seg)
        pltpu.sync_copy(eid_v, eid_hbm)
        cf = c.astype(jnp.float32)
        cm = jnp.where(iota < EXP, cf - (TOK / EXP), 0.0)
        lsv = cm * cm * (LOSS_SCALE / EXP)
        stage_f[...] = jnp.full((16,), jnp.sum(lsv), jnp.float32)
        pltpu.sync_copy(stage_f, loss_hbm)

    pltpu.sync_copy(fill_v, pidx_hbm.at[pl.ds(fb, _SPW)])
    plsc.subcore_barrier()

    cursors = start
    for i in range(_TPW // 16):
        v = lay_v[pl.ds(i * 16, 16)]
        dest = zero16
        for e in range(EXP):
            mask = v == e
            mi = mask.astype(jnp.int32)
            pcs = plsc.cumsum(mi)
            cnt = _splat_sum(mi)
            cur_e = _splat_sum(jnp.where(iota == e, cursors, 0))
            dest = jnp.where(mask, cur_e + pcs - 1, dest)
            cursors = jnp.where(iota == e, cursors + cnt, cursors)
        inv_v[pl.ds(i * 16, 16)] = dest
        row, col = divmod(i, 8)
        dest_v[row, pl.ds(col * 16, 16)] = dest
        tid_v[row, pl.ds(col * 16, 16)] = _splat(base + i * 16) + iota
    pltpu.sync_copy(inv_v, inv_hbm.at[pl.ds(base, _TPW)])
    for k in range(2):
        pltpu.sync_copy(tid_v.at[k], pidx_hbm.at[dest_v.at[k]])


def _dispatch(layer):
    mesh = plsc.VectorSubcoreMesh(core_axis_name="c", subcore_axis_name="s",
                                  num_cores=1, num_subcores=16)
    f = pl.kernel(
        _dispatch_body,
        out_type=(
            jax.ShapeDtypeStruct((NPAD,), jnp.int32),
            jax.ShapeDtypeStruct((TOK,), jnp.int32),
            jax.ShapeDtypeStruct((NEID,), jnp.int32),
            jax.ShapeDtypeStruct((16,), jnp.float32),
        ),
        mesh=mesh,
        scratch_types=[
            pltpu.VMEM((_TPW,), jnp.int32),
            pltpu.VMEM((_TPW,), jnp.int32),
            pltpu.VMEM((2, 128), jnp.int32),
            pltpu.VMEM((2, 128), jnp.int32),
            pltpu.VMEM((128,), jnp.int32),
            pltpu.VMEM((16,), jnp.float32),
            pltpu.VMEM((_SPW,), jnp.int32),
            pltpu.VMEM((NEID,), jnp.int32),
            pltpu.VMEM((16, 128), jnp.int32),
            pltpu.VMEM_SHARED((16, 128), jnp.int32),
        ],
        compiler_params=pltpu.CompilerParams(needs_layout_passes=False),
    )
    return f(layer)


_GB = 16
_GNB = 10
_GBUF = 4


def _gatherx_body(pidx_hbm, xf_hbm, xg_hbm, idx_v, *rest):
    bufs, gsem, wsem = rest[:_GBUF], rest[_GBUF:2 * _GBUF], rest[2 * _GBUF:]
    wid = lax.axis_index("s") * 2 + lax.axis_index("c")
    sb = wid * (NPAD // 32)
    pltpu.sync_copy(pidx_hbm.at[pl.ds(sb, NPAD // 32)], idx_v)
    gd = [pltpu.make_async_copy(xf_hbm.at[idx_v.at[pl.ds(b * _GB, _GB)]],
                                bufs[b % _GBUF], gsem[b % _GBUF])
          for b in range(_GNB)]
    wd = [pltpu.make_async_copy(bufs[b % _GBUF],
                                xg_hbm.at[pl.ds(sb + b * _GB, _GB)],
                                wsem[b % _GBUF]) for b in range(_GNB)]
    for b in range(_GBUF):
        gd[b].start()
    for b in range(_GNB):
        gd[b].wait()
        wd[b].start()
        if b + _GBUF < _GNB:
            wd[b].wait()
            gd[b + _GBUF].start()
    for b in range(_GNB - _GBUF, _GNB):
        wd[b].wait()


def _gatherx(pidx, xf):
    mesh = plsc.VectorSubcoreMesh(core_axis_name="c", subcore_axis_name="s",
                                  num_cores=2, num_subcores=16)
    f = pl.kernel(
        _gatherx_body,
        out_type=jax.ShapeDtypeStruct((NPAD, D), jnp.float32),
        mesh=mesh,
        scratch_types=[pltpu.VMEM((NPAD // 32,), jnp.int32)]
        + [pltpu.VMEM((_GB, D), jnp.float32) for _ in range(_GBUF)]
        + [pltpu.SemaphoreType.DMA for _ in range(2 * _GBUF)],
    )
    return f(pidx, xf)


_OB = 32


def _outgather_body(inv_hbm, outg_hbm, out_hbm, idx_v, rows0, rows1, g0, g1, w0, w1):
    wid = lax.axis_index("s") * 2 + lax.axis_index("c")
    tb = wid * (TOK // 32)
    pltpu.sync_copy(inv_hbm.at[pl.ds(tb, TOK // 32)], idx_v)
    bufs = (rows0, rows1)
    gsem = (g0, g1)
    wsem = (w0, w1)
    gd = [pltpu.make_async_copy(outg_hbm.at[idx_v.at[pl.ds(b * _OB, _OB)]],
                                bufs[b % 2], gsem[b % 2]) for b in range(4)]
    wd = [pltpu.make_async_copy(bufs[b % 2],
                                out_hbm.at[pl.ds(tb + b * _OB, _OB)],
                                wsem[b % 2]) for b in range(4)]
    gd[0].start()
    gd[1].start()
    for b in range(4):
        gd[b].wait()
        wd[b].start()
        if b + 2 < 4:
            wd[b].wait()
            gd[b + 2].start()
    wd[2].wait()
    wd[3].wait()


def _outgather(inv, out_g):
    mesh = plsc.VectorSubcoreMesh(core_axis_name="c", subcore_axis_name="s",
                                  num_cores=2, num_subcores=16)
    f = pl.kernel(
        _outgather_body,
        out_type=jax.ShapeDtypeStruct((TOK, O), jnp.float32),
        mesh=mesh,
        scratch_types=[
            pltpu.VMEM((TOK // 32,), jnp.int32),
            pltpu.VMEM((_OB, O), jnp.float32),
            pltpu.VMEM((_OB, O), jnp.float32),
            pltpu.SemaphoreType.DMA,
            pltpu.SemaphoreType.DMA,
            pltpu.SemaphoreType.DMA,
            pltpu.SemaphoreType.DMA,
        ],
    )
    return f(inv, out_g)


def _gmm_body(eid_ref, x_ref, w_ref, b_ref, o_ref):
    acc = lax.dot_general(x_ref[...], w_ref[0], (((1,), (1,)), ((), ())),
                          preferred_element_type=jnp.float32)
    o_ref[...] = acc + b_ref[0]


def _gmm(eid, xg, Wp, bp3):
    grid_spec = pltpu.PrefetchScalarGridSpec(
        num_scalar_prefetch=1,
        grid=(NT,),
        in_specs=[
            pl.BlockSpec((T, D), lambda i, eid_ref: (i, 0)),
            pl.BlockSpec((1, O, D), lambda i, eid_ref: (eid_ref[i], 0, 0)),
            pl.BlockSpec((1, 1, O), lambda i, eid_ref: (eid_ref[i], 0, 0)),
        ],
        out_specs=pl.BlockSpec((T, O), lambda i, eid_ref: (i, 0)),
    )
    return pl.pallas_call(
        _gmm_body,
        grid_spec=grid_spec,
        out_shape=jax.ShapeDtypeStruct((NPAD, O), jnp.float32),
    )(eid, xg, Wp, bp3)


@jax.jit
def kernel(x, Wp, bp, Wr, br):
    Bb, Cc, Dd = x.shape
    xf = x.reshape(Bb * Cc, Dd)

    layer = _router(xf, Wr, br.reshape(1, EXP))
    pidx, inv, eid, lossv = _dispatch(layer)
    xg = _gatherx(pidx, xf)
    out_g = _gmm(eid[:NT], xg, Wp, bp.reshape(EXP, 1, O))
    out_flat = _outgather(inv, out_g)
    return out_flat.reshape(Bb, Cc, O), lossv[0]
